# baseline (speedup 1.0000x reference)
import functools

import jax
import jax.numpy as jnp
from jax import lax
from jax.experimental import pallas as pl
from jax.experimental.pallas import tpu as pltpu
from jax.experimental.pallas import tpu_sc as plsc

_H = 32
_NW = 32
_BB = 128
_CS = 4
_PR = 129
_PS = 201


def kernel(layer_indices, embedding_table):
    b, s = layer_indices.shape
    n_chunks = s // _CS
    n_grp = n_chunks // 2
    nbt = b // _BB

    mesh = plsc.VectorSubcoreMesh(core_axis_name="c", subcore_axis_name="s")

    @functools.partial(
        pl.kernel,
        mesh=mesh,
        out_type=jax.ShapeDtypeStruct((s * (_H // 8), nbt, 8, _BB), jnp.float32),
        scratch_types=[
            pltpu.VMEM((_BB, _PS), jnp.int32),
            pltpu.VMEM((s, _BB), jnp.int32),
            pltpu.VMEM((2, _CS * _BB, _H), jnp.float32),
            pltpu.VMEM((2, _CS * 4, 1, 8, _PR), jnp.float32),
            pltpu.SemaphoreType.DMA,
            pltpu.SemaphoreType.DMA,
            pltpu.SemaphoreType.DMA,
            pltpu.SemaphoreType.DMA,
        ],
        compiler_params=pltpu.CompilerParams(
            use_tc_tiling_on_sc=False, needs_layout_passes=False),
    )
    def emb(idx_hbm, table_hbm, out_hbm, idxb_v, idxs_v, gbuf, tbuf,
            gsem0, gsem1, osem0, osem1):
        gsems = (gsem0, gsem1)
        osems = (osem0, osem1)
        wid = lax.axis_index("s") * 2 + lax.axis_index("c")

        lanes = lax.iota(jnp.int32, 16)
        bvecs = [g * 16 + lanes for g in range(8)]
        zeros = jnp.zeros((16,), jnp.int32)
        rrvecs = [(q * 16 + lanes) // 8 for q in range(2)]
        hivecs = [lax.rem(q * 16 + lanes, 8) for q in range(2)]

        pltpu.sync_copy(idx_hbm.at[pl.ds(wid * _BB, _BB)],
                        idxb_v.at[:, pl.ds(0, s)])

        def idx_body(si, carry):
            sv = jnp.full((16,), si, jnp.int32)
            for g in range(8):
                v = plsc.load_gather(idxb_v, [bvecs[g], sv])
                idxs_v[si, pl.ds(g * 16, 16)] = v
            return carry

        lax.fori_loop(0, s, idx_body, 0, unroll=False)

        def gather_desc(k, slot, s1):
            return pltpu.make_async_copy(
                table_hbm.at[idxs_v.at[k * _CS + s1]],
                gbuf.at[slot, pl.ds(s1 * _BB, _BB), :],
                gsems[slot])

        def out_desc(k, slot):
            return pltpu.make_async_copy(
                tbuf.at[slot, :, :, :, pl.ds(0, _BB)],
                out_hbm.at[pl.ds(k * _CS * 4, _CS * 4), pl.ds(wid, 1), :, :],
                osems[slot])

        for s1 in range(_CS):
            gather_desc(0, 0, s1).start()

        def transpose_chunk(slot):
            tref = tbuf.at[slot]
            for s1 in range(_CS):
                rr = [rv + s1 * 4 for rv in rrvecs]

                def b_body(bi, cv, s1=s1, rr=rr):
                    for q in range(2):
                        v = gbuf[slot, s1 * _BB + bi, pl.ds(q * 16, 16)]
                        plsc.store_scatter(tref, [rr[q], zeros, hivecs[q], cv], v)
                    return cv + 1

                lax.fori_loop(0, _BB, b_body, jnp.zeros((16,), jnp.int32),
                              unroll=4)

        def group_body(g2, carry):
            for p in range(2):
                k = g2 * 2 + p

                @pl.when(k + 1 < n_chunks)
                def _(p=p, k=k):
                    for s1 in range(_CS):
                        gather_desc(k + 1, 1 - p, s1).start()

                for s1 in range(_CS):
                    gather_desc(k, p, s1).wait()

                @pl.when(k >= 2)
                def _(p=p, k=k):
                    out_desc(k, p).wait()

                transpose_chunk(p)
                out_desc(k, p).start()
            return carry

        lax.fori_loop(0, n_grp, group_body, 0, unroll=False)

        out_desc(n_chunks - 2, 0).wait()
        out_desc(n_chunks - 1, 1).wait()

    out = emb(layer_indices, embedding_table)
    out = out.reshape(s, _H // 8, nbt, 8, _BB)
    return out.transpose(2, 4, 0, 1, 3).reshape(b, s, _H)


# speedup vs baseline: 1.0547x; 1.0547x over previous
import functools

import jax
import jax.numpy as jnp
from jax import lax
from jax.experimental import pallas as pl
from jax.experimental.pallas import tpu as pltpu
from jax.experimental.pallas import tpu_sc as plsc

_H = 32
_NW = 32
_BB = 128
_CS = 2
_PR = 129
_PS = 201


def kernel(layer_indices, embedding_table):
    b, s = layer_indices.shape
    n_chunks = s // _CS
    n_grp = n_chunks // 2
    nbt = b // _BB

    mesh = plsc.VectorSubcoreMesh(core_axis_name="c", subcore_axis_name="s")

    @functools.partial(
        pl.kernel,
        mesh=mesh,
        out_type=jax.ShapeDtypeStruct((s * (_H // 8), nbt, 8, _BB), jnp.float32),
        scratch_types=[
            pltpu.VMEM((_BB, _PS), jnp.int32),
            pltpu.VMEM((s, _BB), jnp.int32),
            pltpu.VMEM((2, _CS * _BB, _H), jnp.float32),
            pltpu.VMEM((2, _CS * 4, 1, 8, _PR), jnp.float32),
            pltpu.SemaphoreType.DMA,
            pltpu.SemaphoreType.DMA,
            pltpu.SemaphoreType.DMA,
            pltpu.SemaphoreType.DMA,
        ],
        compiler_params=pltpu.CompilerParams(
            use_tc_tiling_on_sc=False, needs_layout_passes=False),
    )
    def emb(idx_hbm, table_hbm, out_hbm, idxb_v, idxs_v, gbuf, tbuf,
            gsem0, gsem1, osem0, osem1):
        gsems = (gsem0, gsem1)
        osems = (osem0, osem1)
        wid = lax.axis_index("s") * 2 + lax.axis_index("c")

        lanes = lax.iota(jnp.int32, 16)
        bvecs = [g * 16 + lanes for g in range(8)]
        zeros = jnp.zeros((16,), jnp.int32)
        rrvecs = [(q * 16 + lanes) // 8 for q in range(2)]
        hivecs = [lax.rem(q * 16 + lanes, 8) for q in range(2)]

        pltpu.sync_copy(idx_hbm.at[pl.ds(wid * _BB, _BB)],
                        idxb_v.at[:, pl.ds(0, s)])

        def idx_body(si, carry):
            sv = jnp.full((16,), si, jnp.int32)
            for g in range(8):
                v = plsc.load_gather(idxb_v, [bvecs[g], sv])
                idxs_v[si, pl.ds(g * 16, 16)] = v
            return carry

        lax.fori_loop(0, s, idx_body, 0, unroll=False)

        def gather_desc(k, slot, s1):
            return pltpu.make_async_copy(
                table_hbm.at[idxs_v.at[k * _CS + s1]],
                gbuf.at[slot, pl.ds(s1 * _BB, _BB), :],
                gsems[slot])

        def out_desc(k, slot):
            return pltpu.make_async_copy(
                tbuf.at[slot, :, :, :, pl.ds(0, _BB)],
                out_hbm.at[pl.ds(k * _CS * 4, _CS * 4), pl.ds(wid, 1), :, :],
                osems[slot])

        for s1 in range(_CS):
            gather_desc(0, 0, s1).start()

        def transpose_chunk(k, slot):
            tref = tbuf.at[slot]
            for s1 in range(_CS):
                gather_desc(k, slot, s1).wait()
                rr = [rv + s1 * 4 for rv in rrvecs]

                def b_body(bi, cv, s1=s1, rr=rr):
                    for q in range(2):
                        v = gbuf[slot, s1 * _BB + bi, pl.ds(q * 16, 16)]
                        plsc.store_scatter(tref, [rr[q], zeros, hivecs[q], cv], v)
                    return cv + 1

                lax.fori_loop(0, _BB, b_body, jnp.zeros((16,), jnp.int32),
                              unroll=8)

        def group_body(g2, carry):
            for p in range(2):
                k = g2 * 2 + p

                @pl.when(k + 1 < n_chunks)
                def _(p=p, k=k):
                    for s1 in range(_CS):
                        gather_desc(k + 1, 1 - p, s1).start()

                @pl.when(k >= 2)
                def _(p=p, k=k):
                    out_desc(k, p).wait()

                transpose_chunk(k, p)
                out_desc(k, p).start()
            return carry

        lax.fori_loop(0, n_grp, group_body, 0, unroll=False)

        out_desc(n_chunks - 2, 0).wait()
        out_desc(n_chunks - 1, 1).wait()

    out = emb(layer_indices, embedding_table)
    out = out.reshape(s, _H // 8, nbt, 8, _BB)
    return out.transpose(2, 4, 0, 1, 3).reshape(b, s, _H)


# speedup vs baseline: 1.0580x; 1.0031x over previous
import functools

import jax
import jax.numpy as jnp
from jax import lax
from jax.experimental import pallas as pl
from jax.experimental.pallas import tpu as pltpu
from jax.experimental.pallas import tpu_sc as plsc

_H = 32
_NW = 32
_BB = 128
_CS = 2
_PR = 129
_PS = 201


def kernel(layer_indices, embedding_table):
    b, s = layer_indices.shape
    n_chunks = s // _CS
    n_grp = n_chunks // 2
    nbt = b // _BB

    mesh = plsc.VectorSubcoreMesh(core_axis_name="c", subcore_axis_name="s")

    @functools.partial(
        pl.kernel,
        mesh=mesh,
        out_type=jax.ShapeDtypeStruct((s * (_H // 8), nbt, 8, _BB), jnp.float32),
        scratch_types=[
            pltpu.VMEM((_BB, _PS), jnp.int32),
            pltpu.VMEM((s, _BB), jnp.int32),
            pltpu.VMEM((2, _CS * _BB, _H), jnp.float32),
            pltpu.VMEM((2, _CS * 4, 1, 8, _PR), jnp.float32),
            pltpu.SemaphoreType.DMA,
            pltpu.SemaphoreType.DMA,
            pltpu.SemaphoreType.DMA,
            pltpu.SemaphoreType.DMA,
        ],
        compiler_params=pltpu.CompilerParams(
            use_tc_tiling_on_sc=False, needs_layout_passes=False),
    )
    def emb(idx_hbm, table_hbm, out_hbm, idxb_v, idxs_v, gbuf, tbuf,
            gsem0, gsem1, osem0, osem1):
        gsems = (gsem0, gsem1)
        osems = (osem0, osem1)
        wid = lax.axis_index("s") * 2 + lax.axis_index("c")

        lanes = lax.iota(jnp.int32, 16)
        bvecs = [g * 16 + lanes for g in range(8)]
        zeros = jnp.zeros((16,), jnp.int32)
        rrvecs = [(q * 16 + lanes) // 8 for q in range(2)]
        hivecs = [lax.rem(q * 16 + lanes, 8) for q in range(2)]

        pltpu.sync_copy(idx_hbm.at[pl.ds(wid * _BB, _BB)],
                        idxb_v.at[:, pl.ds(0, s)])

        def idx_body(si, carry):
            sv = jnp.full((16,), si, jnp.int32)
            for g in range(8):
                v = plsc.load_gather(idxb_v, [bvecs[g], sv])
                idxs_v[si, pl.ds(g * 16, 16)] = v
            return carry

        lax.fori_loop(0, s, idx_body, 0, unroll=False)

        def gather_desc(k, slot, s1):
            return pltpu.make_async_copy(
                table_hbm.at[idxs_v.at[k * _CS + s1]],
                gbuf.at[slot, pl.ds(s1 * _BB, _BB), :],
                gsems[slot])

        def out_desc(k, slot):
            return pltpu.make_async_copy(
                tbuf.at[slot, :, :, :, pl.ds(0, _BB)],
                out_hbm.at[pl.ds(k * _CS * 4, _CS * 4), pl.ds(wid, 1), :, :],
                osems[slot])

        for s1 in range(_CS):
            gather_desc(0, 0, s1).start()

        def transpose_chunk(k, slot):
            tref = tbuf.at[slot]
            for s1 in range(_CS):
                gather_desc(k, slot, s1).wait()
                rr = [rv + s1 * 4 for rv in rrvecs]

                def b_body(bi, cv, s1=s1, rr=rr):
                    for q in range(2):
                        v = gbuf[slot, s1 * _BB + bi, pl.ds(q * 16, 16)]
                        plsc.store_scatter(tref, [rr[q], zeros, hivecs[q], cv], v)
                    return cv + 1

                lax.fori_loop(0, _BB, b_body, jnp.zeros((16,), jnp.int32),
                              unroll=16)

        def group_body(g2, carry):
            for p in range(2):
                k = g2 * 2 + p

                @pl.when(k + 1 < n_chunks)
                def _(p=p, k=k):
                    for s1 in range(_CS):
                        gather_desc(k + 1, 1 - p, s1).start()

                @pl.when(k >= 2)
                def _(p=p, k=k):
                    out_desc(k, p).wait()

                transpose_chunk(k, p)
                out_desc(k, p).start()
            return carry

        lax.fori_loop(0, n_grp, group_body, 0, unroll=False)

        out_desc(n_chunks - 2, 0).wait()
        out_desc(n_chunks - 1, 1).wait()

    out = emb(layer_indices, embedding_table)
    out = out.reshape(s, _H // 8, nbt, 8, _BB)
    return out.transpose(2, 4, 0, 1, 3).reshape(b, s, _H)
